# Initial kernel scaffold; baseline (speedup 1.0000x reference)
import jax
import jax.numpy as jnp
from jax.experimental import pallas as pl


def kernel(x, w_g):
    raise NotImplementedError("write your pallas kernel here")


# speedup vs baseline: 1.3761x; 1.3761x over previous
import functools

import jax
import jax.numpy as jnp
from jax.experimental import pallas as pl
from jax.experimental.pallas import tpu as pltpu

TOP_K = 8
BLOCK_T = 1024


def _router_body(x_ref, wt_ref, vals_ref, idxs_ref):
    logits = jax.lax.dot_general(
        x_ref[...], wt_ref[...],
        dimension_numbers=(((1,), (0,)), ((), ())),
        preferred_element_type=jnp.float32,
    )
    n_exp = logits.shape[1]
    iota = jax.lax.broadcasted_iota(jnp.int32, logits.shape, 1)
    cur = logits
    vals = []
    idxs = []
    for _ in range(TOP_K):
        m = jnp.max(cur, axis=1, keepdims=True)
        sel = jnp.min(jnp.where(cur == m, iota, n_exp), axis=1, keepdims=True)
        vals.append(m)
        idxs.append(sel)
        cur = jnp.where(iota == sel, -jnp.inf, cur)
    vals_ref[...] = jnp.concatenate(vals, axis=1)
    idxs_ref[...] = jnp.concatenate(idxs, axis=1)


@jax.jit
def kernel(x, w_g):
    tokens, d = x.shape
    n_exp = w_g.shape[0]
    wt = w_g.T
    grid = (tokens // BLOCK_T,)
    vals, idxs = pl.pallas_call(
        _router_body,
        grid=grid,
        in_specs=[
            pl.BlockSpec((BLOCK_T, d), lambda i: (i, 0)),
            pl.BlockSpec((d, n_exp), lambda i: (0, 0)),
        ],
        out_specs=[
            pl.BlockSpec((BLOCK_T, TOP_K), lambda i: (i, 0)),
            pl.BlockSpec((BLOCK_T, TOP_K), lambda i: (i, 0)),
        ],
        out_shape=[
            jax.ShapeDtypeStruct((tokens, TOP_K), jnp.float32),
            jax.ShapeDtypeStruct((tokens, TOP_K), jnp.int32),
        ],
        compiler_params=pltpu.CompilerParams(
            dimension_semantics=("arbitrary",),
        ),
    )(x, wt)
    return (vals, idxs)
